# baseline (speedup 1.0000x reference)
import jax
import jax.numpy as jnp
from jax.experimental import pallas as pl
from jax.experimental.pallas import tpu as pltpu


def _pass1_kernel(x_ref, adj_ref, W1_ref, b1_ref, W4_ref,
                  s4_ref, abf_ref, s1_ref):
    i = pl.program_id(0)

    @pl.when(i == 0)
    def _compute_support1():
        s1_ref[...] = jnp.dot(x_ref[...], W1_ref[...],
                              preferred_element_type=jnp.float32
                              ).astype(jnp.bfloat16)

    a_bf = adj_ref[...].astype(jnp.bfloat16)
    h = jnp.dot(a_bf, s1_ref[...],
                preferred_element_type=jnp.float32) + b1_ref[...]
    h = jnp.maximum(h, 0.0)
    s4_ref[...] = jnp.dot(h, W4_ref[...], preferred_element_type=jnp.float32)
    abf_ref[...] = a_bf


def _pass2_kernel(abf_ref, s4_ref, b4_ref, out_ref, s4bf_ref):
    i = pl.program_id(0)

    @pl.when(i == 0)
    def _prep():
        s4bf_ref[...] = s4_ref[...].astype(jnp.bfloat16)

    o = jnp.dot(abf_ref[...], s4bf_ref[...],
                preferred_element_type=jnp.float32) + b4_ref[...]
    m = jnp.max(o, axis=1, keepdims=True)
    lse = jnp.log(jnp.sum(jnp.exp(o - m), axis=1, keepdims=True)) + m
    out_ref[...] = o - lse


def kernel(x, adj, W1, b1, W4, b4):
    n, nfeat = x.shape
    nhid = W1.shape[1]
    nclass = W4.shape[1]

    b1_2d = b1.reshape(1, nhid)
    b4_2d = b4.reshape(1, nclass)

    bl1 = 256
    nb1 = pl.cdiv(n, bl1)
    s4, abf = pl.pallas_call(
        _pass1_kernel,
        grid=(nb1,),
        in_specs=[
            pl.BlockSpec((n, nfeat), lambda i: (0, 0)),
            pl.BlockSpec((bl1, n), lambda i: (i, 0)),
            pl.BlockSpec((nfeat, nhid), lambda i: (0, 0)),
            pl.BlockSpec((1, nhid), lambda i: (0, 0)),
            pl.BlockSpec((nhid, nclass), lambda i: (0, 0)),
        ],
        out_specs=[
            pl.BlockSpec((bl1, nclass), lambda i: (i, 0)),
            pl.BlockSpec((bl1, n), lambda i: (i, 0)),
        ],
        out_shape=[
            jax.ShapeDtypeStruct((n, nclass), jnp.float32),
            jax.ShapeDtypeStruct((n, n), jnp.bfloat16),
        ],
        scratch_shapes=[pltpu.VMEM((n, nhid), jnp.bfloat16)],
        compiler_params=pltpu.CompilerParams(
            dimension_semantics=("arbitrary",),
        ),
    )(x, adj, W1, b1_2d, W4)

    bl2 = 256
    nb2 = pl.cdiv(n, bl2)
    out = pl.pallas_call(
        _pass2_kernel,
        grid=(nb2,),
        in_specs=[
            pl.BlockSpec((bl2, n), lambda i: (i, 0)),
            pl.BlockSpec((n, nclass), lambda i: (0, 0)),
            pl.BlockSpec((1, nclass), lambda i: (0, 0)),
        ],
        out_specs=pl.BlockSpec((bl2, nclass), lambda i: (i, 0)),
        out_shape=jax.ShapeDtypeStruct((n, nclass), jnp.float32),
        scratch_shapes=[
            pltpu.VMEM((n, nclass), jnp.bfloat16),
        ],
        compiler_params=pltpu.CompilerParams(
            dimension_semantics=("arbitrary",),
        ),
    )(abf, s4, b4_2d)
    return out
